# Initial kernel scaffold; baseline (speedup 1.0000x reference)
import jax
import jax.numpy as jnp
from jax.experimental import pallas as pl


def kernel(noise_ids, emb, W, b):
    raise NotImplementedError("write your pallas kernel here")


# speedup vs baseline: 7.7375x; 7.7375x over previous
import functools

import jax
import jax.numpy as jnp
from jax.experimental import pallas as pl
from jax.experimental.pallas import tpu as pltpu

N_FEAT = 7
ROWS = 129
EMBED_DIM = 64
HIDDEN = 1024

TOK_BLOCK = 2048


def _fused_body(ids_ref, emb_ref, w_ref, b_ref, out_ref):
    parts = []
    for i in range(N_FEAT):
        ids_col = ids_ref[:, i][:, None]
        iota = jax.lax.broadcasted_iota(jnp.int32, (TOK_BLOCK, ROWS), 1)
        oh = (ids_col == iota).astype(jnp.float32)
        tbl = emb_ref[i * ROWS:(i + 1) * ROWS, :]
        parts.append(jnp.dot(oh, tbl, preferred_element_type=jnp.float32))
    x = jnp.concatenate(parts, axis=-1)
    out_ref[...] = jnp.dot(x, w_ref[...], preferred_element_type=jnp.float32) + b_ref[...]


@jax.jit
def _fused(ids32, emb_flat, W, b):
    n_tok = ids32.shape[0]
    grid = (n_tok // TOK_BLOCK,)
    return pl.pallas_call(
        _fused_body,
        grid=grid,
        in_specs=[
            pl.BlockSpec((TOK_BLOCK, 8), lambda t: (t, 0)),
            pl.BlockSpec((N_FEAT * ROWS, EMBED_DIM), lambda t: (0, 0)),
            pl.BlockSpec((N_FEAT * EMBED_DIM, HIDDEN), lambda t: (0, 0)),
            pl.BlockSpec((1, HIDDEN), lambda t: (0, 0)),
        ],
        out_specs=pl.BlockSpec((TOK_BLOCK, HIDDEN), lambda t: (t, 0)),
        out_shape=jax.ShapeDtypeStruct((n_tok, HIDDEN), jnp.float32),
    )(ids32, emb_flat, W, b)


def kernel(noise_ids, emb, W, b):
    B, S, F = noise_ids.shape
    ids32 = jnp.clip(noise_ids, 0, ROWS - 1).astype(jnp.int32).reshape(B * S, F)
    ids32 = jnp.pad(ids32, ((0, 0), (0, 8 - F)))
    emb_flat = emb.reshape(N_FEAT * ROWS, EMBED_DIM)
    out = _fused(ids32, emb_flat, W, b[None, :])
    return out.reshape(B, S, HIDDEN)
